# Initial kernel scaffold; baseline (speedup 1.0000x reference)
import jax
import jax.numpy as jnp
from jax.experimental import pallas as pl


def kernel(x, edge_index, W0, b0, Wmc, bmc, Wmn, bmn, Wlc, blc, Wln, bln, eps_c, eps_n):
    raise NotImplementedError("write your pallas kernel here")


# speedup vs baseline: 2.6291x; 2.6291x over previous
import functools

import jax
import jax.numpy as jnp
from jax.experimental import pallas as pl
from jax.experimental.pallas import tpu as pltpu


def _mm_kernel(x_ref, w_ref, o_ref):
    o_ref[...] = jnp.dot(x_ref[...], w_ref[...], preferred_element_type=jnp.float32)


def _mm(x, w, block_rows=1000):
    n, d = x.shape
    k = w.shape[1]
    grid = n // block_rows
    return pl.pallas_call(
        _mm_kernel,
        grid=(grid,),
        in_specs=[
            pl.BlockSpec((block_rows, d), lambda i: (i, 0)),
            pl.BlockSpec((d, k), lambda i: (0, 0)),
        ],
        out_specs=pl.BlockSpec((block_rows, k), lambda i: (i, 0)),
        out_shape=jax.ShapeDtypeStruct((n, k), jnp.float32),
    )(x, w)


def _head_kernel(g_ref, w_ref, b_ref, ec_ref, en_ref,
                 muc_ref, mun_ref, lvc_ref, lvn_ref, zc_ref, zn_ref):
    big = jnp.dot(g_ref[...], w_ref[...], preferred_element_type=jnp.float32)
    big = big + b_ref[...]
    h2 = muc_ref.shape[1]
    mu_c = big[:, 0 * h2:1 * h2]
    mu_n = big[:, 1 * h2:2 * h2]
    lv_c = big[:, 2 * h2:3 * h2]
    lv_n = big[:, 3 * h2:4 * h2]
    muc_ref[...] = mu_c
    mun_ref[...] = mu_n
    lvc_ref[...] = lv_c
    lvn_ref[...] = lv_n
    zc_ref[...] = mu_c + ec_ref[...] * jnp.exp(0.5 * lv_c)
    zn_ref[...] = mu_n + en_ref[...] * jnp.exp(0.5 * lv_n)


def _heads(g, wcat, bcat, eps_c, eps_n, block_rows=1000):
    n, h1 = g.shape
    h2 = eps_c.shape[1]
    grid = n // block_rows
    outs = [jax.ShapeDtypeStruct((n, h2), jnp.float32)] * 6
    return pl.pallas_call(
        _head_kernel,
        grid=(grid,),
        in_specs=[
            pl.BlockSpec((block_rows, h1), lambda i: (i, 0)),
            pl.BlockSpec((h1, 4 * h2), lambda i: (0, 0)),
            pl.BlockSpec((1, 4 * h2), lambda i: (0, 0)),
            pl.BlockSpec((block_rows, h2), lambda i: (i, 0)),
            pl.BlockSpec((block_rows, h2), lambda i: (i, 0)),
        ],
        out_specs=[pl.BlockSpec((block_rows, h2), lambda i: (i, 0))] * 6,
        out_shape=outs,
    )(g, wcat, bcat, eps_c, eps_n)


def kernel(x, edge_index, W0, b0, Wmc, bmc, Wmn, bmn, Wlc, blc, Wln, bln, eps_c, eps_n):
    n, d = x.shape
    src, dst = edge_index[0], edge_index[1]
    deg = jnp.ones((n,), jnp.float32).at[dst].add(1.0)
    dinv = 1.0 / jnp.sqrt(deg)
    norm = dinv[src] * dinv[dst]
    dinv2 = (dinv * dinv)[:, None]

    xw = _mm(x, W0)
    s1 = jnp.zeros_like(xw).at[dst].add(xw[src] * norm[:, None])
    h = jax.nn.relu(s1 + dinv2 * xw + b0)
    s2 = jnp.zeros_like(h).at[dst].add(h[src] * norm[:, None])
    g = s2 + dinv2 * h

    wcat = jnp.concatenate([Wmc, Wmn, Wlc, Wln], axis=1)
    bcat = jnp.concatenate([bmc, bmn, blc, bln])[None, :]
    mu_c, mu_n, lv_c, lv_n, z_c, z_n = _heads(g, wcat, bcat, eps_c, eps_n)

    ew_c = jax.nn.relu(jnp.sum(z_c[src] * z_c[dst], axis=1))
    ew_n = jax.nn.relu(jnp.sum(z_n[src] * z_n[dst], axis=1))
    return (ew_c, ew_n, mu_c, mu_n, lv_c, lv_n)


# speedup vs baseline: 3.4740x; 1.3214x over previous
import functools

import jax
import jax.numpy as jnp
from jax import lax
from jax.experimental import pallas as pl
from jax.experimental.pallas import tpu as pltpu
from jax.experimental.pallas import tpu_sc as plsc

_NC = 2
_NS = 16
_NW = _NC * _NS


def _mm_kernel(x_ref, w_ref, o_ref):
    o_ref[...] = jnp.dot(x_ref[...], w_ref[...], preferred_element_type=jnp.float32)


def _mm(x, w, block_rows=1000):
    n, d = x.shape
    k = w.shape[1]
    grid = n // block_rows
    return pl.pallas_call(
        _mm_kernel,
        grid=(grid,),
        in_specs=[
            pl.BlockSpec((block_rows, d), lambda i: (i, 0)),
            pl.BlockSpec((d, k), lambda i: (0, 0)),
        ],
        out_specs=pl.BlockSpec((block_rows, k), lambda i: (i, 0)),
        out_shape=jax.ShapeDtypeStruct((n, k), jnp.float32),
    )(x, w)


def _head_kernel(g_ref, w_ref, b_ref, ec_ref, en_ref,
                 muc_ref, mun_ref, lvc_ref, lvn_ref, zcat_ref):
    big = jnp.dot(g_ref[...], w_ref[...], preferred_element_type=jnp.float32)
    big = big + b_ref[...]
    h2 = muc_ref.shape[1]
    mu_c = big[:, 0 * h2:1 * h2]
    mu_n = big[:, 1 * h2:2 * h2]
    lv_c = big[:, 2 * h2:3 * h2]
    lv_n = big[:, 3 * h2:4 * h2]
    muc_ref[...] = mu_c
    mun_ref[...] = mu_n
    lvc_ref[...] = lv_c
    lvn_ref[...] = lv_n
    zcat_ref[:, 0:h2] = mu_c + ec_ref[...] * jnp.exp(0.5 * lv_c)
    zcat_ref[:, h2:2 * h2] = mu_n + en_ref[...] * jnp.exp(0.5 * lv_n)


def _heads(g, wcat, bcat, eps_c, eps_n, block_rows=1000):
    n, h1 = g.shape
    h2 = eps_c.shape[1]
    grid = n // block_rows
    outs = [jax.ShapeDtypeStruct((n, h2), jnp.float32)] * 4 + [
        jax.ShapeDtypeStruct((n, 2 * h2), jnp.float32)]
    return pl.pallas_call(
        _head_kernel,
        grid=(grid,),
        in_specs=[
            pl.BlockSpec((block_rows, h1), lambda i: (i, 0)),
            pl.BlockSpec((h1, 4 * h2), lambda i: (0, 0)),
            pl.BlockSpec((1, 4 * h2), lambda i: (0, 0)),
            pl.BlockSpec((block_rows, h2), lambda i: (i, 0)),
            pl.BlockSpec((block_rows, h2), lambda i: (i, 0)),
        ],
        out_specs=[pl.BlockSpec((block_rows, h2), lambda i: (i, 0))] * 4 + [
            pl.BlockSpec((block_rows, 2 * h2), lambda i: (i, 0))],
        out_shape=outs,
    )(g, wcat, bcat, eps_c, eps_n)


def _decoder(zcat, srcp, dstp, ep):
    n, h2x2 = zcat.shape
    h2 = h2x2 // 2
    epw = ep // _NW
    C = 16
    nch = epw // C
    mesh = plsc.VectorSubcoreMesh(core_axis_name="c", subcore_axis_name="s")

    @functools.partial(
        pl.kernel,
        mesh=mesh,
        out_type=[jax.ShapeDtypeStruct((ep,), jnp.float32)] * 2,
        scratch_types=[
            pltpu.VMEM((epw,), jnp.int32),
            pltpu.VMEM((epw,), jnp.int32),
            pltpu.VMEM((C, h2x2), jnp.float32),
            pltpu.VMEM((C, h2x2), jnp.float32),
            pltpu.VMEM((C, h2x2), jnp.float32),
            pltpu.VMEM((C, h2x2), jnp.float32),
            pltpu.VMEM((epw,), jnp.float32),
            pltpu.VMEM((epw,), jnp.float32),
            pltpu.SemaphoreType.DMA,
            pltpu.SemaphoreType.DMA,
        ],
    )
    def dec(zcat_hbm, srcp_hbm, dstp_hbm, ewc_hbm, ewn_hbm,
            sbuf, dbuf, zs0, zd0, zs1, zd1, oc, on, sem0, sem1):
        wid = lax.axis_index("s") * _NC + lax.axis_index("c")
        wbase = wid * epw
        pltpu.sync_copy(srcp_hbm.at[pl.ds(wbase, epw)], sbuf)
        pltpu.sync_copy(dstp_hbm.at[pl.ds(wbase, epw)], dbuf)
        lane = lax.iota(jnp.int32, 16)

        def issue(ci, zsb, zdb, sem):
            ci = jnp.minimum(ci, nch - 1)
            pltpu.async_copy(zcat_hbm.at[sbuf.at[pl.ds(ci * C, C)]], zsb, sem)
            pltpu.async_copy(zcat_hbm.at[dbuf.at[pl.ds(ci * C, C)]], zdb, sem)

        def drain(zsb, zdb, sem):
            pltpu.make_async_copy(zcat_hbm.at[sbuf.at[pl.ds(0, C)]], zsb, sem).wait()
            pltpu.make_async_copy(zcat_hbm.at[dbuf.at[pl.ds(0, C)]], zdb, sem).wait()

        perms = [jnp.bitwise_xor(lane, sh) for sh in (8, 4, 2, 1)]

        def lanesum(v):
            for p in perms:
                v = v + v[p]
            return v

        def compute(ci, zsb, zdb):
            def ebody(e, carry):
                vc, vn = carry
                accc = zsb[e, 0:16] * zdb[e, 0:16]
                accn = zsb[e, h2:h2 + 16] * zdb[e, h2:h2 + 16]
                for k in range(1, h2 // 16):
                    accc = accc + zsb[e, 16 * k:16 * k + 16] * zdb[e, 16 * k:16 * k + 16]
                    kk = h2 + 16 * k
                    accn = accn + zsb[e, kk:kk + 16] * zdb[e, kk:kk + 16]
                vc = jnp.where(lane == e, lanesum(accc), vc)
                vn = jnp.where(lane == e, lanesum(accn), vn)
                return (vc, vn)
            z16 = jnp.zeros((16,), jnp.float32)
            vc, vn = lax.fori_loop(0, C, ebody, (z16, z16))
            oc[pl.ds(ci * C, C)] = jnp.maximum(vc, 0.0)
            on[pl.ds(ci * C, C)] = jnp.maximum(vn, 0.0)

        issue(0, zs0, zd0, sem0)

        def body(i0, carry):
            ci = i0 * 2
            issue(ci + 1, zs1, zd1, sem1)
            drain(zs0, zd0, sem0)
            compute(ci, zs0, zd0)
            issue(ci + 2, zs0, zd0, sem0)
            drain(zs1, zd1, sem1)
            compute(ci + 1, zs1, zd1)
            return carry

        lax.fori_loop(0, nch // 2, body, 0)
        drain(zs0, zd0, sem0)
        pltpu.sync_copy(oc, ewc_hbm.at[pl.ds(wbase, epw)])
        pltpu.sync_copy(on, ewn_hbm.at[pl.ds(wbase, epw)])

    return dec(zcat, srcp, dstp)


def kernel(x, edge_index, W0, b0, Wmc, bmc, Wmn, bmn, Wlc, blc, Wln, bln, eps_c, eps_n):
    n, d = x.shape
    src, dst = edge_index[0], edge_index[1]
    deg = jnp.ones((n,), jnp.float32).at[dst].add(1.0)
    dinv = 1.0 / jnp.sqrt(deg)
    norm = dinv[src] * dinv[dst]
    dinv2 = (dinv * dinv)[:, None]

    xw = _mm(x, W0)
    s1 = jnp.zeros_like(xw).at[dst].add(xw[src] * norm[:, None])
    h = jax.nn.relu(s1 + dinv2 * xw + b0)
    s2 = jnp.zeros_like(h).at[dst].add(h[src] * norm[:, None])
    g = s2 + dinv2 * h

    wcat = jnp.concatenate([Wmc, Wmn, Wlc, Wln], axis=1)
    bcat = jnp.concatenate([bmc, bmn, blc, bln])[None, :]
    mu_c, mu_n, lv_c, lv_n, zcat = _heads(g, wcat, bcat, eps_c, eps_n)

    e = src.shape[0]
    ep = ((e + 32 * _NW - 1) // (32 * _NW)) * (32 * _NW)
    srcp = jnp.pad(src, (0, ep - e))
    dstp = jnp.pad(dst, (0, ep - e))
    ewc_p, ewn_p = _decoder(zcat, srcp, dstp, ep)
    return (ewc_p[:e], ewn_p[:e], mu_c, mu_n, lv_c, lv_n)
